# baseline (speedup 1.0000x reference)
import jax
import jax.numpy as jnp
from jax import lax
from jax.experimental import pallas as pl

B, C, H, W = 128, 768, 14, 14
HW = H * W
E, TOPK = 16, 2

BB = 16


def _probe_body(x_ref, noise_ref, w_out, idx_out):
    chunk = x_ref[:, :128]
    r = jnp.sum(chunk, axis=1, keepdims=True)
    w_out[...] = r * jnp.zeros((BB, E), jnp.float32) + noise_ref[...]
    idx_out[...] = jnp.zeros((BB, TOPK), jnp.int32)


@jax.jit
def kernel(x, w1_w, w1_b, w2_w, w2_b, noise):
    xr = x.reshape(B, C * HW)
    grid = (B // BB,)
    w, idx = pl.pallas_call(
        _probe_body,
        grid=grid,
        in_specs=[
            pl.BlockSpec((BB, C * HW), lambda i: (i, 0)),
            pl.BlockSpec((BB, E), lambda i: (i, 0)),
        ],
        out_specs=[
            pl.BlockSpec((BB, E), lambda i: (i, 0)),
            pl.BlockSpec((BB, TOPK), lambda i: (i, 0)),
        ],
        out_shape=[
            jax.ShapeDtypeStruct((B, E), jnp.float32),
            jax.ShapeDtypeStruct((B, TOPK), jnp.int32),
        ],
    )(xr, noise)
    return (w, idx)


# speedup vs baseline: 7.0292x; 7.0292x over previous
import jax
import jax.numpy as jnp
from jax import lax
from jax.experimental import pallas as pl
from jax.experimental.pallas import tpu as pltpu

B, C, H, W = 128, 768, 14, 14
HW = H * W
E, TOPK = 16, 2

KB = 28
NSTEP = HW // KB


def _gate_body(x_ref, wc_ref, bc_ref, noise_ref, w_out, idx_out,
               s_ref, m_ref):
    k = pl.program_id(0)
    xb = x_ref[...]
    ps = jnp.sum(xb, axis=0)
    pm = jnp.max(xb, axis=0)

    @pl.when(k == 0)
    def _init():
        s_ref[...] = ps
        m_ref[...] = pm

    @pl.when(k > 0)
    def _acc():
        s_ref[...] += ps
        m_ref[...] = jnp.maximum(m_ref[...], pm)

    @pl.when(k == NSTEP - 1)
    def _finish():
        f = s_ref[...] * (1.0 / HW) + m_ref[...]
        z = lax.dot_general(
            f.astype(jnp.bfloat16), wc_ref[...],
            dimension_numbers=(((1,), (0,)), ((), ())),
            preferred_element_type=jnp.float32,
        ) + bc_ref[...]

        n1 = z[:, :E]
        n2 = z[:, E:]
        n = n1 + noise_ref[...] * jax.nn.softplus(n2)

        iota = lax.broadcasted_iota(jnp.int32, (B, E), 1)
        v1 = jnp.max(n, axis=1, keepdims=True)
        i1 = jnp.min(jnp.where(n == v1, iota, E), axis=1, keepdims=True)
        masked = jnp.where(iota == i1, -jnp.inf, n)
        v2 = jnp.max(masked, axis=1, keepdims=True)
        i2 = jnp.min(jnp.where(masked == v2, iota, E), axis=1, keepdims=True)

        e2 = jnp.exp(v2 - v1)
        denom = 1.0 + e2
        w_out[...] = jnp.where(
            iota == i1, 1.0 / denom,
            jnp.where(iota == i2, e2 / denom, 0.0))
        idx_out[...] = jnp.concatenate([i1, i2], axis=1)


@jax.jit
def kernel(x, w1_w, w1_b, w2_w, w2_b, noise):
    xt = jnp.transpose(x, (2, 3, 0, 1)).reshape(HW, B, C)
    wc = jnp.concatenate([w1_w, w2_w], axis=0).T.astype(jnp.bfloat16)
    bc = jnp.concatenate([w1_b, w2_b]).reshape(1, 2 * E)

    grid = (NSTEP,)
    w, idx = pl.pallas_call(
        _gate_body,
        grid=grid,
        in_specs=[
            pl.BlockSpec((KB, B, C), lambda k: (k, 0, 0)),
            pl.BlockSpec((C, 2 * E), lambda k: (0, 0)),
            pl.BlockSpec((1, 2 * E), lambda k: (0, 0)),
            pl.BlockSpec((B, E), lambda k: (0, 0)),
        ],
        out_specs=[
            pl.BlockSpec((B, E), lambda k: (0, 0)),
            pl.BlockSpec((B, TOPK), lambda k: (0, 0)),
        ],
        out_shape=[
            jax.ShapeDtypeStruct((B, E), jnp.float32),
            jax.ShapeDtypeStruct((B, TOPK), jnp.int32),
        ],
        scratch_shapes=[
            pltpu.VMEM((B, C), jnp.float32),
            pltpu.VMEM((B, C), jnp.float32),
        ],
    )(xt, wc, bc, noise)
    return (w, idx)


# speedup vs baseline: 7.9192x; 1.1266x over previous
import jax
import jax.numpy as jnp
from jax import lax
from jax.experimental import pallas as pl
from jax.experimental.pallas import tpu as pltpu

B, C, H, W = 128, 768, 14, 14
HW = H * W
E, TOPK = 16, 2

KB = 28
NSTEP = HW // KB


def _gate_body(x_ref, w1_ref, b1_ref, w2_ref, b2_ref, noise_ref,
               w_out, idx_out, s_ref, m_ref):
    k = pl.program_id(0)
    xb = x_ref[...]
    ps = jnp.sum(xb, axis=0)
    pm = jnp.max(xb, axis=0)

    @pl.when(k == 0)
    def _init():
        s_ref[...] = ps
        m_ref[...] = pm

    @pl.when(k > 0)
    def _acc():
        s_ref[...] += ps
        m_ref[...] = jnp.maximum(m_ref[...], pm)

    @pl.when(k == NSTEP - 1)
    def _finish():
        f = s_ref[...] * (1.0 / HW) + m_ref[...]
        fb = f.astype(jnp.bfloat16)
        dn = (((1,), (1,)), ((), ()))
        z1 = lax.dot_general(
            fb, w1_ref[...].astype(jnp.bfloat16), dimension_numbers=dn,
            preferred_element_type=jnp.float32,
        ) + b1_ref[...]
        z2 = lax.dot_general(
            fb, w2_ref[...].astype(jnp.bfloat16), dimension_numbers=dn,
            preferred_element_type=jnp.float32,
        ) + b2_ref[...]

        n1 = z1
        n2 = z2
        n = n1 + noise_ref[...] * jax.nn.softplus(n2)

        iota = lax.broadcasted_iota(jnp.int32, (B, E), 1)
        v1 = jnp.max(n, axis=1, keepdims=True)
        i1 = jnp.min(jnp.where(n == v1, iota, E), axis=1, keepdims=True)
        masked = jnp.where(iota == i1, -jnp.inf, n)
        v2 = jnp.max(masked, axis=1, keepdims=True)
        i2 = jnp.min(jnp.where(masked == v2, iota, E), axis=1, keepdims=True)

        e2 = jnp.exp(v2 - v1)
        denom = 1.0 + e2
        w_out[...] = jnp.where(
            iota == i1, 1.0 / denom,
            jnp.where(iota == i2, e2 / denom, 0.0))
        idx_out[...] = jnp.concatenate([i1, i2], axis=1)


@jax.jit
def kernel(x, w1_w, w1_b, w2_w, w2_b, noise):
    xt = jnp.transpose(x, (2, 3, 0, 1)).reshape(HW, B, C)

    grid = (NSTEP,)
    w, idx = pl.pallas_call(
        _gate_body,
        grid=grid,
        in_specs=[
            pl.BlockSpec((KB, B, C), lambda k: (k, 0, 0)),
            pl.BlockSpec((E, C), lambda k: (0, 0)),
            pl.BlockSpec((1, E), lambda k: (0, 0)),
            pl.BlockSpec((E, C), lambda k: (0, 0)),
            pl.BlockSpec((1, E), lambda k: (0, 0)),
            pl.BlockSpec((B, E), lambda k: (0, 0)),
        ],
        out_specs=[
            pl.BlockSpec((B, E), lambda k: (0, 0)),
            pl.BlockSpec((B, TOPK), lambda k: (0, 0)),
        ],
        out_shape=[
            jax.ShapeDtypeStruct((B, E), jnp.float32),
            jax.ShapeDtypeStruct((B, TOPK), jnp.int32),
        ],
        scratch_shapes=[
            pltpu.VMEM((B, C), jnp.float32),
            pltpu.VMEM((B, C), jnp.float32),
        ],
    )(xt, w1_w, w1_b.reshape(1, E), w2_w, w2_b.reshape(1, E), noise)
    return (w, idx)


# speedup vs baseline: 8.3082x; 1.0491x over previous
import jax
import jax.numpy as jnp
from jax import lax
from jax.experimental import pallas as pl
from jax.experimental.pallas import tpu as pltpu

B, C, H, W = 128, 768, 14, 14
HW = H * W
E, TOPK = 16, 2

KB = 28
NSTEP = HW // KB


def _gate_body(x_ref, w1_ref, b1_ref, w2_ref, b2_ref, noise_ref,
               w_out, idx_out, s_ref, m_ref):
    k = pl.program_id(0)
    xb = x_ref[...]
    ps = jnp.sum(xb, axis=0)
    pm = jnp.max(xb, axis=0)

    @pl.when(k == 0)
    def _init():
        s_ref[...] = ps
        m_ref[...] = pm

    @pl.when(k > 0)
    def _acc():
        s_ref[...] += ps
        m_ref[...] = jnp.maximum(m_ref[...], pm)

    @pl.when(k == NSTEP - 1)
    def _finish():
        f = s_ref[...] * (1.0 / HW) + m_ref[...]
        fb = f.astype(jnp.bfloat16)
        dn = (((1,), (1,)), ((), ()))
        z1 = lax.dot_general(
            fb, w1_ref[...].astype(jnp.bfloat16), dimension_numbers=dn,
            preferred_element_type=jnp.float32,
        ) + b1_ref[...]
        z2 = lax.dot_general(
            fb, w2_ref[...].astype(jnp.bfloat16), dimension_numbers=dn,
            preferred_element_type=jnp.float32,
        ) + b2_ref[...]

        n1 = z1
        n2 = z2
        nz = noise_ref[...].T
        n = n1 + nz * jax.nn.softplus(n2)

        iota = lax.broadcasted_iota(jnp.int32, (B, E), 1)
        v1 = jnp.max(n, axis=1, keepdims=True)
        i1 = jnp.min(jnp.where(n == v1, iota, E), axis=1, keepdims=True)
        masked = jnp.where(iota == i1, -jnp.inf, n)
        v2 = jnp.max(masked, axis=1, keepdims=True)
        i2 = jnp.min(jnp.where(masked == v2, iota, E), axis=1, keepdims=True)

        e2 = jnp.exp(v2 - v1)
        denom = 1.0 + e2
        w_out[...] = jnp.where(
            iota == i1, 1.0 / denom,
            jnp.where(iota == i2, e2 / denom, 0.0))
        idx_out[...] = jnp.concatenate([i1, i2], axis=1)


@jax.jit
def kernel(x, w1_w, w1_b, w2_w, w2_b, noise):
    xt = jnp.transpose(x, (2, 3, 0, 1)).reshape(HW, B, C)

    grid = (NSTEP,)
    w, idx = pl.pallas_call(
        _gate_body,
        grid=grid,
        in_specs=[
            pl.BlockSpec((KB, B, C), lambda k: (k, 0, 0)),
            pl.BlockSpec((E, C), lambda k: (0, 0)),
            pl.BlockSpec((1, E), lambda k: (0, 0)),
            pl.BlockSpec((E, C), lambda k: (0, 0)),
            pl.BlockSpec((1, E), lambda k: (0, 0)),
            pl.BlockSpec((E, B), lambda k: (0, 0)),
        ],
        out_specs=[
            pl.BlockSpec((B, E), lambda k: (0, 0)),
            pl.BlockSpec((B, TOPK), lambda k: (0, 0)),
        ],
        out_shape=[
            jax.ShapeDtypeStruct((B, E), jnp.float32),
            jax.ShapeDtypeStruct((B, TOPK), jnp.int32),
        ],
        scratch_shapes=[
            pltpu.VMEM((B, C), jnp.float32),
            pltpu.VMEM((B, C), jnp.float32),
        ],
    )(xt, w1_w, w1_b.reshape(1, E), w2_w, w2_b.reshape(1, E), noise.T)
    return (w, idx)
